# Initial kernel scaffold; baseline (speedup 1.0000x reference)
import jax
import jax.numpy as jnp
from jax.experimental import pallas as pl


def kernel(edge_attr, u, path_attr, edge_index, path_link_index, W_li, b_li, W_pi, b_pi, W_p2l, b_p2l, W_l2p, b_l2p, Wi_l, Wh_l, bi_l, bh_l, Wi_p, Wh_p, bi_p, bh_p, Wd1, bd1, Wd2, bd2, Wd3, bd3):
    raise NotImplementedError("write your pallas kernel here")


# speedup vs baseline: 4.6143x; 4.6143x over previous
import functools

import jax
import jax.numpy as jnp
from jax import lax
from jax.experimental import pallas as pl
from jax.experimental.pallas import tpu as pltpu
from jax.experimental.pallas import tpu_sc as plsc

H = 64
STEPS = 8
NC = 2
NS = 16
CH = 128
BLK = 2000


@functools.lru_cache(None)
def _seg_gather_kernel(P, K, H0, SH, FCH, NCH):
    mesh = plsc.VectorSubcoreMesh(core_axis_name="c", subcore_axis_name="s")

    def body(x_hbm, z_hbm, src_hbm, dst_hbm, out_hbm,
             src_vm, dst_vm, gbuf, zbuf, shared, sem):
        c = lax.axis_index("c")
        s = lax.axis_index("s")
        pltpu.sync_copy(src_hbm.at[s], src_vm)
        pltpu.sync_copy(dst_hbm.at[c, s], dst_vm)
        pltpu.sync_copy(z_hbm, zbuf)

        @pl.loop(s, NCH, step=NS)
        def _(g):
            pltpu.sync_copy(zbuf, shared.at[pl.ds(g * FCH, FCH)])

        plsc.subcore_barrier()

        @pl.loop(0, K)
        def _(j):
            pltpu.async_copy(x_hbm.at[src_vm.at[j]], gbuf, sem).wait()
            pltpu.sync_copy(gbuf, shared.at[dst_vm.at[j]], add=True)

        plsc.subcore_barrier()

        @pl.loop(s, NCH, step=NS)
        def _(g):
            pltpu.sync_copy(shared.at[pl.ds(g * FCH, FCH)], zbuf)
            pltpu.sync_copy(zbuf, out_hbm.at[pl.ds(c * H0 + g * FCH, FCH)])

    return pl.kernel(
        body,
        out_type=jax.ShapeDtypeStruct((P, H), jnp.float32),
        mesh=mesh,
        scratch_types=[
            pltpu.VMEM((K, CH), jnp.int32),
            pltpu.VMEM((K, CH), jnp.int32),
            pltpu.VMEM((CH, H), jnp.float32),
            pltpu.VMEM((FCH, H), jnp.float32),
            pltpu.VMEM_SHARED((SH, H), jnp.float32),
            pltpu.SemaphoreType.DMA,
        ],
        compiler_params=pltpu.CompilerParams(use_tc_tiling_on_sc=False),
    )


@functools.lru_cache(None)
def _seg_count_kernel(P, K, H0, SH, FCH, NCH):
    mesh = plsc.VectorSubcoreMesh(core_axis_name="c", subcore_axis_name="s")

    def body(ones_hbm, z_hbm, dst_hbm, out_hbm,
             dst_vm, gbuf, zbuf, shared, sem):
        c = lax.axis_index("c")
        s = lax.axis_index("s")
        pltpu.sync_copy(dst_hbm.at[c, s], dst_vm)
        pltpu.sync_copy(ones_hbm, gbuf)
        pltpu.sync_copy(z_hbm, zbuf)

        @pl.loop(s, NCH, step=NS)
        def _(g):
            pltpu.sync_copy(zbuf, shared.at[pl.ds(g * FCH, FCH)])

        plsc.subcore_barrier()

        @pl.loop(0, K)
        def _(j):
            pltpu.sync_copy(gbuf, shared.at[dst_vm.at[j]], add=True)

        plsc.subcore_barrier()

        @pl.loop(s, NCH, step=NS)
        def _(g):
            pltpu.sync_copy(shared.at[pl.ds(g * FCH, FCH)], zbuf)
            pltpu.sync_copy(zbuf, out_hbm.at[pl.ds(c * H0 + g * FCH, FCH)])

    return pl.kernel(
        body,
        out_type=jax.ShapeDtypeStruct((P, H), jnp.float32),
        mesh=mesh,
        scratch_types=[
            pltpu.VMEM((K, CH), jnp.int32),
            pltpu.VMEM((CH, H), jnp.float32),
            pltpu.VMEM((FCH, H), jnp.float32),
            pltpu.VMEM_SHARED((SH, H), jnp.float32),
            pltpu.SemaphoreType.DMA,
        ],
        compiler_params=pltpu.CompilerParams(use_tc_tiling_on_sc=False),
    )


def _full(shape):
    return pl.BlockSpec(shape, lambda i: tuple(0 for _ in shape))


def _encode_link(edge_attr, u2, WeT, WuT, b):
    m = edge_attr.shape[0]

    def body(ea_ref, u_ref, we_ref, wu_ref, b_ref, o_ref):
        ub = jnp.dot(u_ref[...], wu_ref[...], preferred_element_type=jnp.float32)
        x = jnp.dot(ea_ref[...], we_ref[...], preferred_element_type=jnp.float32)
        o_ref[...] = jnp.tanh(x + ub + b_ref[...])

    return pl.pallas_call(
        body,
        out_shape=jax.ShapeDtypeStruct((m, H), jnp.float32),
        grid=(m // BLK,),
        in_specs=[
            pl.BlockSpec((BLK, edge_attr.shape[1]), lambda i: (i, 0)),
            _full(u2.shape), _full(WeT.shape), _full(WuT.shape), _full(b.shape),
        ],
        out_specs=pl.BlockSpec((BLK, H), lambda i: (i, 0)),
    )(edge_attr, u2, WeT, WuT, b)


def _encode_path(path_attr, WT, b):
    p = path_attr.shape[0]

    def body(pa_ref, w_ref, b_ref, o_ref):
        x = jnp.dot(pa_ref[...], w_ref[...], preferred_element_type=jnp.float32)
        o_ref[...] = jnp.tanh(x + b_ref[...])

    return pl.pallas_call(
        body,
        out_shape=jax.ShapeDtypeStruct((p, H), jnp.float32),
        grid=(p // BLK,),
        in_specs=[pl.BlockSpec((BLK, path_attr.shape[1]), lambda i: (i, 0)),
                  _full(WT.shape), _full(b.shape)],
        out_specs=pl.BlockSpec((BLK, H), lambda i: (i, 0)),
    )(path_attr, WT, b)


def _relu_linear(x, WT, b):
    n = x.shape[0]

    def body(x_ref, w_ref, b_ref, o_ref):
        y = jnp.dot(x_ref[...], w_ref[...], preferred_element_type=jnp.float32)
        o_ref[...] = jnp.maximum(y + b_ref[...], 0.0)

    return pl.pallas_call(
        body,
        out_shape=jax.ShapeDtypeStruct((n, H), jnp.float32),
        grid=(n // BLK,),
        in_specs=[pl.BlockSpec((BLK, H), lambda i: (i, 0)),
                  _full(WT.shape), _full(b.shape)],
        out_specs=pl.BlockSpec((BLK, H), lambda i: (i, 0)),
    )(x, WT, b)


def _gru(agg, cnt, h, WiT, WhT, bi, bh, sqrt_scale):
    n = h.shape[0]

    def body(a_ref, c_ref, h_ref, wi_ref, wh_ref, bi_ref, bh_ref, o_ref):
        cv = jnp.maximum(c_ref[...], 1.0)
        scale = lax.rsqrt(cv) if sqrt_scale else 1.0 / cv
        x = a_ref[...] * scale
        h0 = h_ref[...]
        gi = jnp.dot(x, wi_ref[...], preferred_element_type=jnp.float32) + bi_ref[...]
        gh = jnp.dot(h0, wh_ref[...], preferred_element_type=jnp.float32) + bh_ref[...]
        r = jax.nn.sigmoid(gi[:, :H] + gh[:, :H])
        z = jax.nn.sigmoid(gi[:, H:2 * H] + gh[:, H:2 * H])
        nn = jnp.tanh(gi[:, 2 * H:] + r * gh[:, 2 * H:])
        o_ref[...] = (1.0 - z) * nn + z * h0

    return pl.pallas_call(
        body,
        out_shape=jax.ShapeDtypeStruct((n, H), jnp.float32),
        grid=(n // BLK,),
        in_specs=[
            pl.BlockSpec((BLK, H), lambda i: (i, 0)),
            pl.BlockSpec((BLK, 1), lambda i: (i, 0)),
            pl.BlockSpec((BLK, H), lambda i: (i, 0)),
            _full(WiT.shape), _full(WhT.shape), _full(bi.shape), _full(bh.shape),
        ],
        out_specs=pl.BlockSpec((BLK, H), lambda i: (i, 0)),
    )(agg, cnt, h, WiT, WhT, bi, bh)


def _gru_zero_input(h, WhT, bi, bh, steps):
    n = h.shape[0]

    def body(h_ref, wh_ref, bi_ref, bh_ref, o_ref):
        h0 = h_ref[...]
        gi = bi_ref[...]
        for _ in range(steps):
            gh = jnp.dot(h0, wh_ref[...], preferred_element_type=jnp.float32) + bh_ref[...]
            r = jax.nn.sigmoid(gi[:, :H] + gh[:, :H])
            z = jax.nn.sigmoid(gi[:, H:2 * H] + gh[:, H:2 * H])
            nn = jnp.tanh(gi[:, 2 * H:] + r * gh[:, 2 * H:])
            h0 = (1.0 - z) * nn + z * h0
        o_ref[...] = h0

    return pl.pallas_call(
        body,
        out_shape=jax.ShapeDtypeStruct((n, H), jnp.float32),
        grid=(n // BLK,),
        in_specs=[pl.BlockSpec((BLK, H), lambda i: (i, 0)),
                  _full(WhT.shape), _full(bi.shape), _full(bh.shape)],
        out_specs=pl.BlockSpec((BLK, H), lambda i: (i, 0)),
    )(h, WhT, bi, bh)


def _decoder(h, ea, W1hT, W1eT, b1, W2T, b2, W3T, b3):
    n = h.shape[0]
    nt = W3T.shape[1]

    def body(h_ref, ea_ref, w1h_ref, w1e_ref, b1_ref, w2_ref, b2_ref,
             w3_ref, b3_ref, o_ref):
        x = jnp.dot(h_ref[...], w1h_ref[...], preferred_element_type=jnp.float32)
        x = x + jnp.dot(ea_ref[...], w1e_ref[...], preferred_element_type=jnp.float32)
        x = jnp.maximum(x + b1_ref[...], 0.0)
        x = jnp.maximum(
            jnp.dot(x, w2_ref[...], preferred_element_type=jnp.float32) + b2_ref[...], 0.0)
        o_ref[...] = jnp.dot(x, w3_ref[...], preferred_element_type=jnp.float32) + b3_ref[...]

    return pl.pallas_call(
        body,
        out_shape=jax.ShapeDtypeStruct((n, nt), jnp.float32),
        grid=(n // BLK,),
        in_specs=[
            pl.BlockSpec((BLK, H), lambda i: (i, 0)),
            pl.BlockSpec((BLK, ea.shape[1]), lambda i: (i, 0)),
            _full(W1hT.shape), _full(W1eT.shape), _full(b1.shape),
            _full(W2T.shape), _full(b2.shape), _full(W3T.shape), _full(b3.shape),
        ],
        out_specs=pl.BlockSpec((BLK, nt), lambda i: (i, 0)),
    )(h, ea, W1hT, W1eT, b1, W2T, b2, W3T, b3)


def kernel(edge_attr, u, path_attr, edge_index, path_link_index,
           W_li, b_li, W_pi, b_pi, W_p2l, b_p2l, W_l2p, b_l2p,
           Wi_l, Wh_l, bi_l, bh_l, Wi_p, Wh_p, bi_p, bh_p,
           Wd1, bd1, Wd2, bd2, Wd3, bd3):
    m = edge_attr.shape[0]
    p = path_attr.shape[0]
    q = path_link_index.shape[1]
    ed = edge_attr.shape[1]

    K = -(-q // (NS * CH))
    qpad = NS * CH * K
    H0 = p // NC
    FCH = next(d for d in range(min(H0, 256) // 8 * 8, 0, -8) if H0 % d == 0)
    NCH = H0 // FCH
    SH = H0 + 16
    TRASH = H0 + 8

    path_ids = path_link_index[0].astype(jnp.int32)
    link_ids = path_link_index[1].astype(jnp.int32)

    def prep(src, dst):
        npad = qpad - q
        srcp = jnp.concatenate([src, jnp.zeros((npad,), jnp.int32)])
        dstp = jnp.concatenate([dst, jnp.full((npad,), p, jnp.int32)])
        src_g = srcp.reshape(NS, K, CH)
        parts = []
        for c in range(NC):
            lo = c * H0
            parts.append(jnp.where((dstp >= lo) & (dstp < lo + H0),
                                   dstp - lo, TRASH).reshape(NS, K, CH))
        return src_g, jnp.stack(parts)

    src1, dst1 = prep(path_ids, link_ids)
    src2, dst2 = prep(link_ids, path_ids)

    zeros_fl = jnp.zeros((FCH, H), jnp.float32)
    ones_ch = jnp.ones((CH, H), jnp.float32)

    seg = _seg_gather_kernel(p, K, H0, SH, FCH, NCH)
    segcnt = _seg_count_kernel(p, K, H0, SH, FCH, NCH)

    cnt_link = segcnt(ones_ch, zeros_fl, dst1)[:, :1]
    cnt_path = segcnt(ones_ch, zeros_fl, dst2)[:, :1]

    u2 = u[None, :]
    WeT = W_li[:, :ed].T
    WuT = W_li[:, ed:].T
    b_li2 = b_li[None, :]
    WpiT = W_pi.T
    b_pi2 = b_pi[None, :]
    Wp2lT = W_p2l.T
    b_p2l2 = b_p2l[None, :]
    Wl2pT = W_l2p.T
    b_l2p2 = b_l2p[None, :]
    WilT, WhlT = Wi_l.T, Wh_l.T
    bil2, bhl2 = bi_l[None, :], bh_l[None, :]
    WipT, WhpT = Wi_p.T, Wh_p.T
    bip2, bhp2 = bi_p[None, :], bh_p[None, :]
    W1hT = Wd1[:, :H].T
    W1eT = Wd1[:, H:].T
    b1 = bd1[None, :]
    W2T = Wd2.T
    b2 = bd2[None, :]
    W3T = Wd3.T
    b3 = bd3[None, :]

    h_link0 = _encode_link(edge_attr, u2, WeT, WuT, b_li2)
    h_path = _encode_path(path_attr, WpiT, b_pi2)

    h_lo = h_link0[:p]
    out_hi = None
    if m > p:
        h_hi = _gru_zero_input(h_link0[p:], WhlT, bil2, bhl2, STEPS)

    for _ in range(STEPS):
        msg_p = _relu_linear(h_path, Wp2lT, b_p2l2)
        agg_l = seg(msg_p, zeros_fl, src1, dst1)
        h_lo = _gru(agg_l, cnt_link, h_lo, WilT, WhlT, bil2, bhl2, True)
        msg_l = _relu_linear(h_lo, Wl2pT, b_l2p2)
        agg_p = seg(msg_l, zeros_fl, src2, dst2)
        h_path = _gru(agg_p, cnt_path, h_path, WipT, WhpT, bip2, bhp2, False)

    out_lo = _decoder(h_lo, edge_attr[:p], W1hT, W1eT, b1, W2T, b2, W3T, b3)
    if m > p:
        out_hi = _decoder(h_hi, edge_attr[p:], W1hT, W1eT, b1, W2T, b2, W3T, b3)
        return jnp.concatenate([out_lo, out_hi], axis=0)
    return out_lo
